# baseline (speedup 1.0000x reference)
import jax
import jax.numpy as jnp
from jax import lax
from jax.experimental import pallas as pl
from jax.experimental.pallas import tpu as pltpu
from jax.experimental.pallas import tpu_sc as plsc

N_NODES = 10000
D = 128
E = 320000
N_LAYERS = 3

NC = 2
NS = 16
HALF = D // NC
RPT = 640
N_PAD = NS * RPT
C = 128
G = 8
NBUF = 4
EPS = E // NS
NSUPER = -(-EPS // (G * C))
EPS_PAD = NSUPER * G * C
LANES = 16
RC = 128


def _sc_body(ego0_h, col_h, row_h, val_h, out_h,
             col2, row2, val2, gbufs, sbufs, ego_sp, acc,
             stage_sem, gsems, ssems):
    c = lax.axis_index("c")
    s = lax.axis_index("s")
    base = s * RPT
    zero16f = jnp.zeros((LANES,), jnp.float32)
    tmp, tmp2 = sbufs[0], sbufs[1]
    tmp_bf, tmp_bf2 = gbufs[0], gbufs[1]

    def _pack_row(dst_bf, r, vecs):
        for h in range(HALF // (2 * LANES)):
            p = plsc.pack(vecs[2 * h], vecs[2 * h + 1],
                          format=plsc.PackFormat.INTERLEAVED)
            dst_bf[r, pl.ds(h * 2 * LANES, 2 * LANES)] = p

    def _unpack_row(src_bf, r):
        vecs = []
        for h in range(HALF // (2 * LANES)):
            ab = src_bf[r, pl.ds(h * 2 * LANES, 2 * LANES)]
            a, b2 = plsc.unpack(ab, format=plsc.PackFormat.INTERLEAVED)
            vecs += [a, b2]
        return vecs

    def _zero_tmp2(r, carry):
        for d in range(HALF // LANES):
            tmp2[r, pl.ds(d * LANES, LANES)] = zero16f
        return carry

    lax.fori_loop(0, RC, _zero_tmp2, 0)

    def _init(q, carry):
        rb = base + q * RC
        pltpu.sync_copy(ego0_h.at[pl.ds(c * N_PAD + rb, RC)], tmp)

        def _to_bf(r, carry2):
            vecs = [tmp[r, pl.ds(d * LANES, LANES)]
                    for d in range(HALF // LANES)]
            _pack_row(tmp_bf, r, vecs)
            return carry2

        lax.fori_loop(0, RC, _to_bf, 0)
        pltpu.sync_copy(tmp_bf, ego_sp.at[pl.ds(rb, RC)])
        pltpu.sync_copy(tmp2, acc.at[pl.ds(rb, RC)])
        return carry

    lax.fori_loop(0, RPT // RC, _init, 0)
    plsc.subcore_barrier()

    def _layer(k, carry):
        def _super(jsc, carry2):
            a1 = pltpu.async_copy(col_h.at[s, jsc], col2, stage_sem)
            a2 = pltpu.async_copy(row_h.at[s, jsc], row2, stage_sem)
            a3 = pltpu.async_copy(val_h.at[s, jsc], val2, stage_sem)
            a1.wait(); a2.wait(); a3.wait()

            gd = [None] * NBUF
            sd = [None] * NBUF
            gd[0] = pltpu.async_copy(ego_sp.at[col2.at[0]], gbufs[0],
                                     gsems[0])
            gd[1] = pltpu.async_copy(ego_sp.at[col2.at[1]], gbufs[1],
                                     gsems[1])
            for g in range(G):
                b = g % NBUF
                if g + 2 < G:
                    nb = (g + 2) % NBUF
                    if sd[nb] is not None:
                        sd[nb].wait()
                        sd[nb] = None
                    gd[nb] = pltpu.async_copy(ego_sp.at[col2.at[g + 2]],
                                              gbufs[nb], gsems[nb])
                gd[b].wait()

                @plsc.parallel_loop(0, C // LANES, unroll=2)
                def _scale(gi, _g=g, _gbf=gbufs[b], _sb=sbufs[b]):
                    e0 = gi * LANES
                    v16 = val2[_g, pl.ds(e0, LANES)]
                    for j in range(LANES):
                        v = v16[j]
                        vecs = _unpack_row(_gbf, e0 + j)
                        for d in range(HALF // LANES):
                            sl = pl.ds(d * LANES, LANES)
                            _sb[e0 + j, sl] = vecs[d] * v

                sd[b] = pltpu.async_copy(sbufs[b], acc.at[row2.at[g]],
                                         ssems[b], add=True)
            for b in range(NBUF):
                if sd[b] is not None:
                    sd[b].wait()
            return carry2

        lax.fori_loop(0, NSUPER, _super, 0)
        plsc.subcore_barrier()

        def _readout(q, carry2):
            rb = base + q * RC
            pltpu.sync_copy(acc.at[pl.ds(rb, RC)], tmp)

            @pl.when(k == 0)
            def _():
                @plsc.parallel_loop(0, RC, unroll=2)
                def _copy_bf(r):
                    vecs = [tmp[r, pl.ds(d * LANES, LANES)]
                            for d in range(HALF // LANES)]
                    _pack_row(tmp_bf, r, vecs)

                pltpu.sync_copy(tmp_bf, ego_sp.at[pl.ds(rb, RC)])

            @pl.when(k == 1)
            def _():
                pltpu.sync_copy(ego_sp.at[pl.ds(rb, RC)], tmp_bf2)

                @plsc.parallel_loop(0, RC, unroll=2)
                def _diff(r):
                    old = _unpack_row(tmp_bf2, r)
                    vecs = []
                    for d in range(HALF // LANES):
                        sl = pl.ds(d * LANES, LANES)
                        vecs.append(tmp[r, sl] - old[d])
                    _pack_row(tmp_bf, r, vecs)

                pltpu.sync_copy(tmp_bf, ego_sp.at[pl.ds(rb, RC)])

            @pl.when(k == N_LAYERS - 1)
            def _():
                inv = jnp.float32(1.0 / N_LAYERS)

                @plsc.parallel_loop(0, RC, unroll=2)
                def _mean(r):
                    for d in range(HALF // LANES):
                        sl = pl.ds(d * LANES, LANES)
                        tmp[r, sl] = tmp[r, sl] * inv

                pltpu.sync_copy(tmp, out_h.at[c, pl.ds(rb, RC)])
            return carry2

        lax.fori_loop(0, RPT // RC, _readout, 0)
        plsc.subcore_barrier()
        return carry

    lax.fori_loop(0, N_LAYERS, _layer, 0)


@jax.jit
def _run(ego0, col, row, val):
    mesh = plsc.VectorSubcoreMesh(core_axis_name="c", subcore_axis_name="s",
                                  num_cores=NC, num_subcores=NS)
    f = pl.kernel(
        _sc_body,
        out_type=jax.ShapeDtypeStruct((NC, N_PAD, HALF), jnp.float32),
        mesh=mesh,
        compiler_params=pltpu.CompilerParams(use_tc_tiling_on_sc=False,
                                             needs_layout_passes=False),
        scratch_types=[
            pltpu.VMEM((G, C), jnp.int32),
            pltpu.VMEM((G, C), jnp.int32),
            pltpu.VMEM((G, C), jnp.float32),
            [pltpu.VMEM((C, HALF), jnp.bfloat16) for _ in range(NBUF)],
            [pltpu.VMEM((C, HALF), jnp.float32) for _ in range(NBUF)],
            pltpu.VMEM_SHARED((N_PAD, HALF), jnp.bfloat16),
            pltpu.VMEM_SHARED((N_PAD, HALF), jnp.float32),
            pltpu.SemaphoreType.DMA,
            [pltpu.SemaphoreType.DMA for _ in range(NBUF)],
            [pltpu.SemaphoreType.DMA for _ in range(NBUF)],
        ],
    )
    return f(ego0, col, row, val)


def kernel(user_emb, item_emb, edge_vals, edge_index):
    ego0 = jnp.concatenate([user_emb, item_emb], axis=0)
    ego0 = jnp.pad(ego0, ((0, N_PAD - N_NODES), (0, 0)))
    ego0 = ego0.reshape(N_PAD, NC, HALF).transpose(1, 0, 2).reshape(NC * N_PAD, HALF)

    pad = NS * EPS_PAD - E
    col = jnp.pad(edge_index[1], (0, pad)).reshape(NS, NSUPER, G, C)
    row = jnp.pad(edge_index[0], (0, pad)).reshape(NS, NSUPER, G, C)
    val = jnp.pad(edge_vals, (0, pad)).reshape(NS, NSUPER, G, C)

    out = _run(ego0, col, row, val)
    final = out[:, :N_NODES, :].transpose(1, 0, 2).reshape(N_NODES, D)
    return (final[:N_NODES // 2], final[N_NODES // 2:])
